# baseline (speedup 1.0000x reference)
import jax
import jax.numpy as jnp
from jax import lax
from jax.experimental import pallas as pl
from jax.experimental.pallas import tpu as pltpu
from jax.experimental.pallas import tpu_sc as plsc

VOCAB = 1000000
HIDDEN = 32
EMB = 128
BATCH = 16384
HIST = 50
NTOK = BATCH * HIST


_INFO = plsc.get_sparse_core_info()
_NC = _INFO.num_cores
_NS = _INFO.num_subcores
_NW = _NC * _NS
_ROWS_PER_W = NTOK // _NW
_GSTEP = 128
_NSTEP = 8
_CHUNK = _GSTEP * _NSTEP
_NCHUNK = _ROWS_PER_W // _CHUNK


def _sc_gather_body(idx_hbm, table_hbm, h_hbm, idx_v, rows_v, sem):
    wid = lax.axis_index("s") * _NC + lax.axis_index("c")
    base = wid * _ROWS_PER_W

    def chunk(c, carry):
        off = base + c * _CHUNK
        pltpu.sync_copy(idx_hbm.at[pl.ds(off, _CHUNK)], idx_v)
        copies = []
        for j in range(_NSTEP):
            copies.append(pltpu.async_copy(
                table_hbm.at[idx_v.at[pl.ds(j * _GSTEP, _GSTEP)]],
                rows_v.at[pl.ds(j * _GSTEP, _GSTEP)],
                sem,
            ))
        for cp in copies:
            cp.wait()
        pltpu.sync_copy(rows_v, h_hbm.at[pl.ds(off, _CHUNK)])
        return carry

    lax.fori_loop(0, _NCHUNK, chunk, 0)


def _sc_gather(x_flat, u_weight):
    mesh = plsc.VectorSubcoreMesh(core_axis_name="c", subcore_axis_name="s")
    k = pl.kernel(
        _sc_gather_body,
        out_type=jax.ShapeDtypeStruct((NTOK, HIDDEN), jnp.float32),
        mesh=mesh,
        scratch_types=[
            pltpu.VMEM((_CHUNK,), jnp.int32),
            pltpu.VMEM((_CHUNK, HIDDEN), jnp.float32),
            pltpu.SemaphoreType.DMA,
        ],
        compiler_params=pltpu.CompilerParams(use_tc_tiling_on_sc=False),
    )
    return k(x_flat, u_weight)


_MB = 16
_MTOK = _MB * HIST
_MROW = _MTOK // 4


def _mm_body(hp_ref, v4_ref, b4_ref, o_ref):
    y = (
        jnp.dot(hp_ref[...], v4_ref[...], preferred_element_type=jnp.float32)
        + b4_ref[...]
    )
    o_ref[...] = y.reshape(_MB, HIST, EMB)


def _tc_project(hp, v_weight, v_bias):
    v4 = jnp.kron(jnp.eye(4, dtype=jnp.float32), v_weight)
    b4 = jnp.tile(v_bias, 4).reshape(1, 4 * EMB)
    return pl.pallas_call(
        _mm_body,
        grid=(BATCH // _MB,),
        in_specs=[
            pl.BlockSpec((_MROW, EMB), lambda i: (i, 0)),
            pl.BlockSpec((EMB, 4 * EMB), lambda i: (0, 0)),
            pl.BlockSpec((1, 4 * EMB), lambda i: (0, 0)),
        ],
        out_specs=pl.BlockSpec((_MB, HIST, EMB), lambda i: (i, 0, 0)),
        out_shape=jax.ShapeDtypeStruct((BATCH, HIST, EMB), jnp.float32),
    )(hp, v4, b4)


@jax.jit
def kernel(x, u_weight, v_weight, v_bias):
    x_flat = x.reshape(-1).astype(jnp.int32)
    h = _sc_gather(x_flat, u_weight)
    hp = h.reshape(NTOK // 4, 4 * HIDDEN)
    return _tc_project(hp, v_weight, v_bias)
